# baseline (speedup 1.0000x reference)
import functools

import jax
import jax.numpy as jnp
from jax import lax
from jax.experimental import pallas as pl
from jax.experimental.pallas import tpu as pltpu
from jax.experimental.pallas import tpu_sc as plsc

HIDDEN = 64
SUBROWS = 8
N_TOKENS = 800
N_TOKENS_PAD = 1024
VB = 2048
RG = 250

_NC, _NS = 2, 16
_NW = _NC * _NS
_B_PER_W = N_TOKENS_PAD // _NW


def _repack_body(in_ref, out_ref):
    x = in_ref[...]
    out_ref[:, :HIDDEN] = x.reshape(RG * SUBROWS, HIDDEN)


@functools.cache
def _make_sc_gather(n_rows):
    @functools.partial(
        pl.kernel,
        mesh=plsc.VectorSubcoreMesh(core_axis_name="c", subcore_axis_name="s"),
        out_type=jax.ShapeDtypeStruct((N_TOKENS_PAD, 2 * HIDDEN), jnp.float32),
        scratch_types=[
            pltpu.VMEM((_B_PER_W,), jnp.int32),
            pltpu.VMEM((_B_PER_W, 2 * HIDDEN), jnp.float32),
            pltpu.SemaphoreType.DMA,
        ],
    )
    def _sc_gather(idx_hbm, table_hbm, out_hbm, idx_v, rows_v, sem):
        wid = lax.axis_index("s") * _NC + lax.axis_index("c")
        base = wid * _B_PER_W
        pltpu.sync_copy(idx_hbm.at[pl.ds(base, _B_PER_W)], idx_v)
        pltpu.async_copy(table_hbm.at[idx_v], rows_v, sem).wait()
        pltpu.sync_copy(rows_v, out_hbm.at[pl.ds(base, _B_PER_W)])

    return _sc_gather


def _head_body(h2_ref, w_ref, out_ref):
    out_ref[...] = lax.dot_general(
        h2_ref[:, :HIDDEN], w_ref[...],
        (((1,), (1,)), ((), ())),
        preferred_element_type=jnp.float32,
    )


def kernel(input_ids, attention_mask, embed, lm_head_w):
    del attention_mask
    B, T = input_ids.shape
    V = lm_head_w.shape[0]

    ids = jnp.reshape(input_ids, (-1,)).astype(jnp.int32)
    idx = jnp.pad(ids, (0, N_TOKENS_PAD - N_TOKENS))
    table3 = jnp.reshape(embed, (V // SUBROWS, SUBROWS, HIDDEN))

    table128 = pl.pallas_call(
        _repack_body,
        grid=(V // (RG * SUBROWS),),
        in_specs=[pl.BlockSpec((RG, SUBROWS, HIDDEN), lambda i: (i, 0, 0))],
        out_specs=pl.BlockSpec((RG * SUBROWS, 2 * HIDDEN), lambda i: (i, 0)),
        out_shape=jax.ShapeDtypeStruct((V, 2 * HIDDEN), jnp.float32),
    )(table3)

    h2 = _make_sc_gather(V)(idx, table128)

    n_vb = pl.cdiv(V, VB)
    logits = pl.pallas_call(
        _head_body,
        grid=(n_vb,),
        in_specs=[
            pl.BlockSpec((N_TOKENS, 2 * HIDDEN), lambda i: (0, 0)),
            pl.BlockSpec((VB, HIDDEN), lambda i: (i, 0)),
        ],
        out_specs=pl.BlockSpec((N_TOKENS, VB), lambda i: (0, i)),
        out_shape=jax.ShapeDtypeStruct((N_TOKENS, V), jnp.float32),
    )(h2, lm_head_w)

    return jnp.reshape(logits, (B, T, V))


# speedup vs baseline: 3.6517x; 3.6517x over previous
import functools

import jax
import jax.numpy as jnp
from jax import lax
from jax.experimental import pallas as pl
from jax.experimental.pallas import tpu as pltpu
from jax.experimental.pallas import tpu_sc as plsc

HIDDEN = 64
N_TOKENS = 800
N_TOKENS_PAD = 1024
VB = 2048
RB = 1024

_NC, _NS = 2, 16
_NW = _NC * _NS
_B_PER_W = N_TOKENS_PAD // _NW


def _repack_body(in_ref, out_ref):
    out_ref[:, :HIDDEN] = in_ref[...].T


@functools.cache
def _make_sc_gather(n_rows):
    @functools.partial(
        pl.kernel,
        mesh=plsc.VectorSubcoreMesh(core_axis_name="c", subcore_axis_name="s"),
        out_type=jax.ShapeDtypeStruct((N_TOKENS_PAD, 2 * HIDDEN), jnp.float32),
        scratch_types=[
            pltpu.VMEM((_B_PER_W,), jnp.int32),
            pltpu.VMEM((_B_PER_W, 2 * HIDDEN), jnp.float32),
            pltpu.SemaphoreType.DMA,
        ],
    )
    def _sc_gather(idx_hbm, table_hbm, out_hbm, idx_v, rows_v, sem):
        wid = lax.axis_index("s") * _NC + lax.axis_index("c")
        base = wid * _B_PER_W
        pltpu.sync_copy(idx_hbm.at[pl.ds(base, _B_PER_W)], idx_v)
        pltpu.async_copy(table_hbm.at[idx_v], rows_v, sem).wait()
        pltpu.sync_copy(rows_v, out_hbm.at[pl.ds(base, _B_PER_W)])

    return _sc_gather


def _head_body(h2_ref, wt_ref, out_ref):
    out_ref[...] = lax.dot_general(
        h2_ref[:, :HIDDEN], wt_ref[...],
        (((1,), (0,)), ((), ())),
        preferred_element_type=jnp.float32,
    )


def kernel(input_ids, attention_mask, embed, lm_head_w):
    del attention_mask
    B, T = input_ids.shape
    V = lm_head_w.shape[0]

    ids = jnp.reshape(input_ids.T, (-1,)).astype(jnp.int32)
    idx = jnp.pad(ids, (0, N_TOKENS_PAD - N_TOKENS))
    embed_t = embed.T
    w_t = lm_head_w.T

    table128 = pl.pallas_call(
        _repack_body,
        grid=(pl.cdiv(V, RB),),
        in_specs=[pl.BlockSpec((HIDDEN, RB), lambda i: (0, i))],
        out_specs=pl.BlockSpec((RB, 2 * HIDDEN), lambda i: (i, 0)),
        out_shape=jax.ShapeDtypeStruct((V, 2 * HIDDEN), jnp.float32),
    )(embed_t)

    h2 = _make_sc_gather(V)(idx, table128)

    n_vb = pl.cdiv(V, VB)
    logits = pl.pallas_call(
        _head_body,
        grid=(n_vb,),
        in_specs=[
            pl.BlockSpec((N_TOKENS, 2 * HIDDEN), lambda i: (0, 0)),
            pl.BlockSpec((HIDDEN, VB), lambda i: (0, i)),
        ],
        out_specs=pl.BlockSpec((N_TOKENS, VB), lambda i: (0, i)),
        out_shape=jax.ShapeDtypeStruct((N_TOKENS, V), jnp.float32),
    )(h2, w_t)

    return jnp.transpose(jnp.reshape(logits, (T, B, V)), (1, 0, 2))


# speedup vs baseline: 3.7149x; 1.0173x over previous
import functools

import jax
import jax.numpy as jnp
from jax import lax
from jax.experimental import pallas as pl
from jax.experimental.pallas import tpu as pltpu
from jax.experimental.pallas import tpu_sc as plsc

HIDDEN = 64
N_TOKENS = 800
N_TOKENS_PAD = 1024
VB = 4096
RB = 1024

_NC, _NS = 2, 16
_NW = _NC * _NS
_B_PER_W = N_TOKENS_PAD // _NW


def _repack_body(in_ref, out_ref):
    out_ref[:, :HIDDEN] = in_ref[...].T


@functools.cache
def _make_sc_gather(n_rows):
    @functools.partial(
        pl.kernel,
        mesh=plsc.VectorSubcoreMesh(core_axis_name="c", subcore_axis_name="s"),
        out_type=jax.ShapeDtypeStruct((N_TOKENS_PAD, 2 * HIDDEN), jnp.float32),
        scratch_types=[
            pltpu.VMEM((_B_PER_W,), jnp.int32),
            pltpu.VMEM((_B_PER_W, 2 * HIDDEN), jnp.float32),
            pltpu.SemaphoreType.DMA,
        ],
    )
    def _sc_gather(idx_hbm, table_hbm, out_hbm, idx_v, rows_v, sem):
        wid = lax.axis_index("s") * _NC + lax.axis_index("c")
        base = wid * _B_PER_W
        pltpu.sync_copy(idx_hbm.at[pl.ds(base, _B_PER_W)], idx_v)
        pltpu.async_copy(table_hbm.at[idx_v], rows_v, sem).wait()
        pltpu.sync_copy(rows_v, out_hbm.at[pl.ds(base, _B_PER_W)])

    return _sc_gather


def _head_body(h2_ref, wt_ref, out_ref):
    out_ref[...] = lax.dot_general(
        h2_ref[:, :HIDDEN], wt_ref[...],
        (((1,), (0,)), ((), ())),
        preferred_element_type=jnp.float32,
    )


def kernel(input_ids, attention_mask, embed, lm_head_w):
    del attention_mask
    B, T = input_ids.shape
    V = lm_head_w.shape[0]

    ids = jnp.reshape(input_ids.T, (-1,)).astype(jnp.int32)
    idx = jnp.pad(ids, (0, N_TOKENS_PAD - N_TOKENS))
    embed_t = embed.T
    w_t = lm_head_w.T

    table128 = pl.pallas_call(
        _repack_body,
        grid=(pl.cdiv(V, RB),),
        in_specs=[pl.BlockSpec((HIDDEN, RB), lambda i: (0, i))],
        out_specs=pl.BlockSpec((RB, 2 * HIDDEN), lambda i: (i, 0)),
        out_shape=jax.ShapeDtypeStruct((V, 2 * HIDDEN), jnp.float32),
    )(embed_t)

    h2 = _make_sc_gather(V)(idx, table128)

    n_vb = pl.cdiv(V, VB)
    logits = pl.pallas_call(
        _head_body,
        grid=(n_vb,),
        in_specs=[
            pl.BlockSpec((N_TOKENS, 2 * HIDDEN), lambda i: (0, 0)),
            pl.BlockSpec((HIDDEN, VB), lambda i: (0, i)),
        ],
        out_specs=pl.BlockSpec((N_TOKENS, VB), lambda i: (0, i)),
        out_shape=jax.ShapeDtypeStruct((N_TOKENS, V), jnp.float32),
    )(h2, w_t)

    return jnp.transpose(jnp.reshape(logits, (T, B, V)), (1, 0, 2))


# speedup vs baseline: 3.7252x; 1.0028x over previous
import functools

import jax
import jax.numpy as jnp
from jax import lax
from jax.experimental import pallas as pl
from jax.experimental.pallas import tpu as pltpu
from jax.experimental.pallas import tpu_sc as plsc

HIDDEN = 64
N_TOKENS = 800
N_TOKENS_PAD = 1024
VB = 8192
RB = 1024

_NC, _NS = 2, 16
_NW = _NC * _NS
_B_PER_W = N_TOKENS_PAD // _NW


def _repack_body(in_ref, out_ref):
    out_ref[:, :HIDDEN] = in_ref[...].T


@functools.cache
def _make_sc_gather(n_rows):
    @functools.partial(
        pl.kernel,
        mesh=plsc.VectorSubcoreMesh(core_axis_name="c", subcore_axis_name="s"),
        out_type=jax.ShapeDtypeStruct((N_TOKENS_PAD, 2 * HIDDEN), jnp.float32),
        scratch_types=[
            pltpu.VMEM((_B_PER_W,), jnp.int32),
            pltpu.VMEM((_B_PER_W, 2 * HIDDEN), jnp.float32),
            pltpu.SemaphoreType.DMA,
        ],
    )
    def _sc_gather(idx_hbm, table_hbm, out_hbm, idx_v, rows_v, sem):
        wid = lax.axis_index("s") * _NC + lax.axis_index("c")
        base = wid * _B_PER_W
        pltpu.sync_copy(idx_hbm.at[pl.ds(base, _B_PER_W)], idx_v)
        pltpu.async_copy(table_hbm.at[idx_v], rows_v, sem).wait()
        pltpu.sync_copy(rows_v, out_hbm.at[pl.ds(base, _B_PER_W)])

    return _sc_gather


def _head_body(h2_ref, wt_ref, out_ref):
    out_ref[...] = lax.dot_general(
        h2_ref[:, :HIDDEN], wt_ref[...],
        (((1,), (0,)), ((), ())),
        preferred_element_type=jnp.float32,
    )


def kernel(input_ids, attention_mask, embed, lm_head_w):
    del attention_mask
    B, T = input_ids.shape
    V = lm_head_w.shape[0]

    ids = jnp.reshape(input_ids.T, (-1,)).astype(jnp.int32)
    idx = jnp.pad(ids, (0, N_TOKENS_PAD - N_TOKENS))
    embed_t = embed.T
    w_t = lm_head_w.T

    table128 = pl.pallas_call(
        _repack_body,
        grid=(pl.cdiv(V, RB),),
        in_specs=[pl.BlockSpec((HIDDEN, RB), lambda i: (0, i))],
        out_specs=pl.BlockSpec((RB, 2 * HIDDEN), lambda i: (i, 0)),
        out_shape=jax.ShapeDtypeStruct((V, 2 * HIDDEN), jnp.float32),
    )(embed_t)

    h2 = _make_sc_gather(V)(idx, table128)

    n_vb = pl.cdiv(V, VB)
    logits = pl.pallas_call(
        _head_body,
        grid=(n_vb,),
        in_specs=[
            pl.BlockSpec((N_TOKENS, 2 * HIDDEN), lambda i: (0, 0)),
            pl.BlockSpec((HIDDEN, VB), lambda i: (0, i)),
        ],
        out_specs=pl.BlockSpec((N_TOKENS, VB), lambda i: (0, i)),
        out_shape=jax.ShapeDtypeStruct((N_TOKENS, V), jnp.float32),
    )(h2, w_t)

    return jnp.transpose(jnp.reshape(logits, (T, B, V)), (1, 0, 2))


# speedup vs baseline: 4.7956x; 1.2873x over previous
import functools

import jax
import jax.numpy as jnp
from jax import lax
from jax.experimental import pallas as pl
from jax.experimental.pallas import tpu as pltpu
from jax.experimental.pallas import tpu_sc as plsc

HIDDEN = 64
N_TOKENS = 800
N_TOKENS_PAD = 1024
VB = 4096
VH = 50176
RB = 7168

_NC, _NS = 2, 16
_NW = _NC * _NS
_B_PER_W = N_TOKENS_PAD // _NW


def _repack_body(lo_ref, hi_ref, out_ref):
    out_ref[:, :HIDDEN] = lo_ref[...].T
    out_ref[:, HIDDEN:] = hi_ref[...].T


@functools.cache
def _make_sc_gather(n_rows):
    @functools.partial(
        pl.kernel,
        mesh=plsc.VectorSubcoreMesh(core_axis_name="c", subcore_axis_name="s"),
        out_type=jax.ShapeDtypeStruct((N_TOKENS_PAD, 2 * HIDDEN), jnp.float32),
        scratch_types=[
            pltpu.VMEM((_B_PER_W,), jnp.int32),
            pltpu.VMEM((_B_PER_W, 2 * HIDDEN), jnp.float32),
            pltpu.SemaphoreType.DMA,
        ],
    )
    def _sc_gather(idx_hbm, table_hbm, out_hbm, idx_v, rows_v, sem):
        wid = lax.axis_index("s") * _NC + lax.axis_index("c")
        base = wid * _B_PER_W
        pltpu.sync_copy(idx_hbm.at[pl.ds(base, _B_PER_W)], idx_v)
        pltpu.async_copy(table_hbm.at[idx_v], rows_v, sem).wait()
        pltpu.sync_copy(rows_v, out_hbm.at[pl.ds(base, _B_PER_W)])

    return _sc_gather


def _head_body(h2_ref, m_ref, wt_ref, out_ref):
    m = m_ref[...]
    h = h2_ref[:, :HIDDEN] * (1.0 - m) + h2_ref[:, HIDDEN:] * m
    out_ref[...] = lax.dot_general(
        h, wt_ref[...],
        (((1,), (0,)), ((), ())),
        preferred_element_type=jnp.float32,
    )


def kernel(input_ids, attention_mask, embed, lm_head_w):
    del attention_mask
    B, T = input_ids.shape
    V = lm_head_w.shape[0]

    ids = jnp.reshape(input_ids.T, (-1,)).astype(jnp.int32)
    hi = ids >= VH
    rows = jnp.where(hi, ids - VH, ids)
    idx = jnp.pad(rows, (0, N_TOKENS_PAD - N_TOKENS))
    half = hi.astype(jnp.float32).reshape(N_TOKENS, 1)
    embed_t = embed.T
    w_t = lm_head_w.T

    table128 = pl.pallas_call(
        _repack_body,
        grid=(VH // RB,),
        in_specs=[
            pl.BlockSpec((HIDDEN, RB), lambda i: (0, i)),
            pl.BlockSpec((HIDDEN, RB), lambda i: (0, VH // RB + i)),
        ],
        out_specs=pl.BlockSpec((RB, 2 * HIDDEN), lambda i: (i, 0)),
        out_shape=jax.ShapeDtypeStruct((VH, 2 * HIDDEN), jnp.float32),
    )(embed_t, embed_t)

    h2 = _make_sc_gather(VH)(idx, table128)

    n_vb = pl.cdiv(V, VB)
    logits = pl.pallas_call(
        _head_body,
        grid=(n_vb,),
        in_specs=[
            pl.BlockSpec((N_TOKENS, 2 * HIDDEN), lambda i: (0, 0)),
            pl.BlockSpec((N_TOKENS, 1), lambda i: (0, 0)),
            pl.BlockSpec((HIDDEN, VB), lambda i: (0, i)),
        ],
        out_specs=pl.BlockSpec((N_TOKENS, VB), lambda i: (0, i)),
        out_shape=jax.ShapeDtypeStruct((N_TOKENS, V), jnp.float32),
    )(h2, half, w_t)

    return jnp.transpose(jnp.reshape(logits, (T, B, V)), (1, 0, 2))
